# baseline (speedup 1.0000x reference)
import functools

import jax
import jax.numpy as jnp
from jax import lax
from jax.experimental import pallas as pl
from jax.experimental.pallas import tpu as pltpu
from jax.experimental.pallas import tpu_sc as plsc

F32 = jnp.float32


def _prep_conv1(w):
    wt = jnp.transpose(w, (2, 3, 1, 0))
    taps = []
    for dy in range(2):
        for dx in range(2):
            cols = []
            for v in range(2):
                for u in range(2):
                    m = jnp.zeros((4, 4, 3, 128), F32)
                    for p4y in range(4):
                        ky = 4 * dy + p4y - 2 * v
                        if not 0 <= ky < 4:
                            continue
                        for p4x in range(4):
                            kx = 4 * dx + p4x - 2 * u
                            if 0 <= kx < 4:
                                m = m.at[p4y, p4x].set(wt[ky, kx])
                    cols.append(m.reshape(48, 128))
            taps.append(jnp.concatenate(cols, axis=1))
    return jnp.stack(taps)


def _prep_conv2(w):
    wt = jnp.transpose(w, (2, 3, 1, 0))
    mats = []
    for py in range(2):
        for px in range(2):
            for dy in range(2):
                for dx in range(2):
                    mats.append(wt[2 * dy + py, 2 * dx + px])
    return jnp.stack(mats)


def _prep_s1(w):
    return jnp.transpose(w, (2, 3, 1, 0)).reshape(9, w.shape[1], w.shape[0])


_TCONV_VALID = {}
for _s in range(3):
    for _sx in range(3):
        _TCONV_VALID[(_s, _sx)] = [
            (py, px) for py in range(2) if 0 <= _s - py <= 1
            for px in range(2) if 0 <= _sx - px <= 1]


def _prep_tconv(wt):
    mats = {}
    for (s, sx), valid in _TCONV_VALID.items():
        cols = [wt[:, :, 3 - py - 2 * (s - py), 3 - px - 2 * (sx - px)]
                for py, px in valid]
        mats[(s, sx)] = jnp.concatenate(cols, axis=1)
    return mats


def _prep_tconv2_packed(wt):
    Cin, Cout = wt.shape[0], wt.shape[1]
    zero = jnp.zeros((Cin, Cout), F32)
    mats = []
    for a in range(4):
        for b in range(4):
            cols = []
            for t in range(2):
                for u in range(2):
                    for qy in range(2):
                        for qx in range(2):
                            ey, ex = a - t, b - u
                            ok = (0 <= ey <= 2 and 0 <= ex <= 2
                                  and ey - qy in (0, 1) and ex - qx in (0, 1))
                            cols.append(
                                wt[:, :, 3 + qy - 2 * ey, 3 + qx - 2 * ex]
                                if ok else zero)
            mats.append(jnp.concatenate(cols, axis=1))
    return jnp.stack(mats)


def _enc_body(xs_ref, w1_ref, b1_ref, w2_ref, b2_ref, w3_ref, b3_ref,
              cbt_ref, idx_ref, loss_ref, p1_ref, h2_ref):
    n = pl.program_id(0)

    @pl.when(n == 0)
    def _():
        p1_ref[...] = jnp.zeros((2, 2, 58, 64, 128), F32)
        h2_ref[...] = jnp.zeros((58, 64, 128), F32)
        loss_ref[...] = jnp.zeros((1, 1), F32)

    acc1 = jnp.zeros((3136, 512), F32)
    for dy in range(2):
        for dx in range(2):
            xs = xs_ref[0, dy:dy + 56, dx:dx + 56, :].reshape(3136, 48)
            acc1 = acc1 + jnp.dot(xs, w1_ref[dy * 2 + dx],
                                  preferred_element_type=F32)
    acc1 = jnp.maximum(acc1 + b1_ref[:], 0.0)
    for v in range(2):
        for u in range(2):
            p = (v * 2 + u) * 128
            p1_ref[v, u, 1:57, 1:57, :] = acc1[:, p:p + 128].reshape(56, 56, 128)

    acc = jnp.zeros((3136, 128), F32)
    for py in range(2):
        for px in range(2):
            for dy in range(2):
                for dx in range(2):
                    xs = p1_ref[py ^ 1, px ^ 1, dy + py:dy + py + 56,
                                dx + px:dx + px + 56, :].reshape(3136, 128)
                    acc = acc + jnp.dot(xs, w2_ref[((py * 2 + px) * 2 + dy) * 2 + dx],
                                        preferred_element_type=F32)
    acc = jnp.maximum(acc + b2_ref[:], 0.0)
    h2_ref[1:57, 1:57, :] = acc.reshape(56, 56, 128)

    acc = jnp.zeros((3136, 32), F32)
    for ky in range(3):
        for kx in range(3):
            xs = h2_ref[ky:ky + 56, kx:kx + 56, :].reshape(3136, 128)
            acc = acc + jnp.dot(xs, w3_ref[ky * 3 + kx], preferred_element_type=F32)
    z = acc + b3_ref[:]

    cb_sq = jnp.sum(cbt_ref[:] ** 2, axis=0, keepdims=True)
    loss_part = jnp.sum(z * z).reshape(1, 1)
    for c in range(4):
        zc = z[c * 784:(c + 1) * 784, :]
        scores = cb_sq - 2.0 * jnp.dot(zc, cbt_ref[:], preferred_element_type=F32)
        iota = jax.lax.broadcasted_iota(jnp.int32, (784, 1024), 1)
        m = jnp.min(scores, axis=1, keepdims=True)
        idx = jnp.min(jnp.where(scores == m, iota, 1024), axis=1, keepdims=True)
        loss_part = loss_part + jnp.sum(m).reshape(1, 1)
        idx_ref[0, c * 784:(c + 1) * 784, :] = idx
    loss_ref[:] = loss_ref[:] + loss_part


def _sc_codebook_gather(codebook, idx_flat):
    B = idx_flat.shape[0]
    D = codebook.shape[1]
    info = plsc.get_sparse_core_info()
    NW = info.num_cores * info.num_subcores
    n_chunks = 2
    assert B % (8 * NW * n_chunks) == 0
    b_per_c = B // (NW * n_chunks)
    mesh = plsc.VectorSubcoreMesh(core_axis_name="c", subcore_axis_name="s")

    @functools.partial(
        pl.kernel, mesh=mesh,
        out_type=jax.ShapeDtypeStruct((B, D), F32),
        scratch_types=[
            pltpu.VMEM((b_per_c,), jnp.int32),
            pltpu.VMEM((b_per_c, D), F32),
            pltpu.SemaphoreType.DMA,
        ],
    )
    def k(table_hbm, idx_hbm, out_hbm, idx_v, rows_v, sem):
        wid = lax.axis_index("s") * info.num_cores + lax.axis_index("c")
        for c in range(n_chunks):
            base = (wid * n_chunks + c) * b_per_c
            pltpu.sync_copy(idx_hbm.at[pl.ds(base, b_per_c)], idx_v)
            pltpu.async_copy(table_hbm.at[idx_v], rows_v, sem).wait()
            pltpu.sync_copy(rows_v, out_hbm.at[pl.ds(base, b_per_c)])

    return k(codebook, idx_flat)


_T2 = {0: (1, 0), 1: (0, 1), 2: (1, 1), 3: (0, 2)}
_CORNERS = [(0, 0), (0, 2), (2, 0), (2, 2)]
_EDGES = [(0, 1), (1, 0), (1, 2), (2, 1)]


def _dec_body(q_ref, w5_ref, b5_ref, w6k_ref, w6e_ref, w6c_ref, b6_ref,
              w7_ref, b7_ref, o_ref, qp_ref, h5_ref, p6_ref):
    n = pl.program_id(0)

    @pl.when(n == 0)
    def _():
        qp_ref[...] = jnp.zeros((58, 64, 32), F32)
        h5_ref[...] = jnp.zeros((58, 64, 128), F32)
        p6_ref[...] = jnp.zeros((2, 2, 58, 64, 128), F32)

    qp_ref[1:57, 1:57, :] = q_ref[0, :, :, 0:32]

    acc = jnp.zeros((3136, 128), F32)
    for ky in range(3):
        for kx in range(3):
            xs = qp_ref[ky:ky + 56, kx:kx + 56, :].reshape(3136, 32)
            acc = acc + jnp.dot(xs, w5_ref[ky * 3 + kx],
                                preferred_element_type=F32)
    acc = jnp.maximum(acc + b5_ref[:], 0.0)
    h5_ref[1:57, 1:57, :] = acc.reshape(56, 56, 128)

    accs6 = [[jnp.zeros((3136, 128), F32) for _ in range(2)] for _ in range(2)]
    for s in range(3):
        for sx in range(3):
            valid = _TCONV_VALID[(s, sx)]
            if (s, sx) == (1, 1):
                w = w6c_ref[:]
            elif len(valid) == 2:
                w = w6e_ref[_EDGES.index((s, sx))]
            else:
                w = w6k_ref[_CORNERS.index((s, sx))]
            xs = h5_ref[s:s + 56, sx:sx + 56, :].reshape(3136, 128)
            r = jnp.dot(xs, w, preferred_element_type=F32)
            for i, (py, px) in enumerate(valid):
                accs6[py][px] = accs6[py][px] + r[:, i * 128:(i + 1) * 128]
    for py in range(2):
        for px in range(2):
            acc = jnp.maximum(accs6[py][px] + b6_ref[:], 0.0)
            p6_ref[py, px, 1:57, 1:57, :] = acc.reshape(56, 56, 128)

    acc48 = jnp.zeros((3136, 48), F32)
    for a in range(4):
        wy, jy = _T2[a]
        for b in range(4):
            wx, jx = _T2[b]
            xs = p6_ref[wy, wx, jy:jy + 56, jx:jx + 56, :].reshape(3136, 128)
            acc48 = acc48 + jnp.dot(xs, w7_ref[a * 4 + b],
                                    preferred_element_type=F32)
    acc48 = acc48 + b7_ref[:]
    o_ref[0] = acc48.reshape(56, 56, 48)


def kernel(x, enc_w1, enc_b1, enc_w2, enc_b2, enc_w3, enc_b3, codebook,
           dec_w1, dec_b1, dec_wt1, dec_bt1, dec_wt2, dec_bt2):
    N = x.shape[0]

    w1 = _prep_conv1(enc_w1)
    w2 = _prep_conv2(enc_w2)
    w3 = _prep_s1(enc_w3)
    w5 = _prep_s1(dec_w1)
    w6m = _prep_tconv(dec_wt1)
    w6k = jnp.stack([w6m[c] for c in _CORNERS])
    w6e = jnp.stack([w6m[e] for e in _EDGES])
    w6c = w6m[(1, 1)]
    w7 = _prep_tconv2_packed(dec_wt2)
    b1 = jnp.tile(enc_b1, 4).reshape(1, -1)
    b2 = enc_b2.reshape(1, -1)
    b3 = enc_b3.reshape(1, -1)
    b5 = dec_b1.reshape(1, -1)
    b6 = dec_bt1.reshape(1, -1)
    b7 = jnp.tile(dec_bt2, 16).reshape(1, -1)

    xh = jnp.transpose(x, (0, 2, 3, 1))
    xp = jnp.pad(xh, ((0, 0), (1, 3), (1, 3), (0, 0)))
    xs4 = xp.reshape(N, 57, 4, 57, 4, 3).transpose(0, 1, 3, 2, 4, 5)
    xs4 = jnp.pad(xs4.reshape(N, 57, 57, 48), ((0, 0), (0, 0), (0, 7), (0, 0)))

    idx3, loss_sum = pl.pallas_call(
        _enc_body,
        grid=(N,),
        in_specs=[
            pl.BlockSpec((1, 57, 64, 48), lambda n: (n, 0, 0, 0)),
            pl.BlockSpec((4, 48, 512), lambda n: (0, 0, 0)),
            pl.BlockSpec((1, 512), lambda n: (0, 0)),
            pl.BlockSpec((16, 128, 128), lambda n: (0, 0, 0)),
            pl.BlockSpec((1, 128), lambda n: (0, 0)),
            pl.BlockSpec((9, 128, 32), lambda n: (0, 0, 0)),
            pl.BlockSpec((1, 32), lambda n: (0, 0)),
            pl.BlockSpec((32, 1024), lambda n: (0, 0)),
        ],
        out_specs=[
            pl.BlockSpec((1, 3136, 1), lambda n: (n, 0, 0)),
            pl.BlockSpec((1, 1), lambda n: (0, 0)),
        ],
        out_shape=[
            jax.ShapeDtypeStruct((N, 3136, 1), jnp.int32),
            jax.ShapeDtypeStruct((1, 1), F32),
        ],
        scratch_shapes=[
            pltpu.VMEM((2, 2, 58, 64, 128), F32),
            pltpu.VMEM((58, 64, 128), F32),
        ],
    )(xs4, w1, b1, w2, b2, w3, b3, jnp.transpose(codebook))

    cb_pad = jnp.pad(codebook, ((0, 0), (0, 96)))
    q_flat = _sc_codebook_gather(cb_pad, idx3.reshape(N * 3136))
    q = q_flat.reshape(N, 56, 56, 128)

    out = pl.pallas_call(
        _dec_body,
        grid=(N,),
        in_specs=[
            pl.BlockSpec((1, 56, 56, 128), lambda n: (n, 0, 0, 0)),
            pl.BlockSpec((9, 32, 128), lambda n: (0, 0, 0)),
            pl.BlockSpec((1, 128), lambda n: (0, 0)),
            pl.BlockSpec((4, 128, 128), lambda n: (0, 0, 0)),
            pl.BlockSpec((4, 128, 256), lambda n: (0, 0, 0)),
            pl.BlockSpec((128, 512), lambda n: (0, 0)),
            pl.BlockSpec((1, 128), lambda n: (0, 0)),
            pl.BlockSpec((16, 128, 48), lambda n: (0, 0, 0)),
            pl.BlockSpec((1, 48), lambda n: (0, 0)),
        ],
        out_specs=pl.BlockSpec((1, 56, 56, 48), lambda n: (n, 0, 0, 0)),
        out_shape=jax.ShapeDtypeStruct((N, 56, 56, 48), F32),
        scratch_shapes=[
            pltpu.VMEM((58, 64, 32), F32),
            pltpu.VMEM((58, 64, 128), F32),
            pltpu.VMEM((2, 2, 58, 64, 128), F32),
        ],
    )(q, w5, b5, w6k, w6e, w6c, b6, w7, b7)

    o = out.reshape(N, 56, 56, 2, 2, 2, 2, 3)
    o = o.transpose(0, 1, 3, 5, 2, 4, 6, 7).reshape(N, 224, 224, 3)
    o = o.transpose(0, 3, 1, 2)

    loss = (1.25 / (N * 56 * 56 * 32)) * loss_sum[0, 0]
    return (o, loss)
